# Initial kernel scaffold; baseline (speedup 1.0000x reference)
import jax
import jax.numpy as jnp
from jax.experimental import pallas as pl


def kernel(input, offset):
    raise NotImplementedError("write your pallas kernel here")


# speedup vs baseline: 5.6709x; 5.6709x over previous
import functools

import jax
import jax.numpy as jnp
from jax import lax
from jax.experimental import pallas as pl
from jax.experimental.pallas import tpu as pltpu
from jax.experimental.pallas import tpu_sc as plsc

H = 224
W = 224
P = H * W
K = 9
C = 96
N = K * P
NW = 32
NP = N // NW
CH = 112
NCHUNK = NP // CH
NV = C // 16


def _transpose_in(inp2d):
    PB = 512

    def body(x_ref, o_ref):
        o_ref[...] = x_ref[...].T

    return pl.pallas_call(
        body,
        grid=(P // PB,),
        in_specs=[pl.BlockSpec((C, PB), lambda j: (0, j))],
        out_specs=pl.BlockSpec((PB, C), lambda j: (j, 0)),
        out_shape=jax.ShapeDtypeStruct((P, C), jnp.float32),
    )(inp2d)


def _prep(off):
    R = 56

    def body(o_ref, idx_ref, wgt_ref):
        k = pl.program_id(0)
        r = pl.program_id(1)
        ki = (k // 3).astype(jnp.float32)
        kj = (k % 3).astype(jnp.float32)
        ho = lax.broadcasted_iota(jnp.int32, (R, W), 0) + r * R
        wo = lax.broadcasted_iota(jnp.int32, (R, W), 1)
        y = (ho - 1).astype(jnp.float32) + ki + o_ref[0]
        x = (wo - 1).astype(jnp.float32) + kj + o_ref[1]
        y0 = jnp.floor(y)
        x0 = jnp.floor(x)
        ly = y - y0
        lx = x - x0
        hy = 1.0 - ly
        hx = 1.0 - lx
        y1 = y0 + 1.0
        x1 = x0 + 1.0
        corners = ((y0, x0, hy, hx), (y0, x1, hy, lx),
                   (y1, x0, ly, hx), (y1, x1, ly, lx))
        for ci, (yf, xf, wy, wx) in enumerate(corners):
            valid = ((yf >= 0.0) & (yf <= float(H - 1))
                     & (xf >= 0.0) & (xf <= float(W - 1)))
            yc = jnp.clip(yf, 0.0, float(H - 1))
            xc = jnp.clip(xf, 0.0, float(W - 1))
            idx_ref[ci, 0] = (yc * float(W) + xc).astype(jnp.int32)
            wgt_ref[ci, 0] = wy * wx * valid.astype(jnp.float32)

    return pl.pallas_call(
        body,
        grid=(K, H // R),
        in_specs=[pl.BlockSpec((2, R, W), lambda k, r: (k, r, 0))],
        out_specs=[
            pl.BlockSpec((4, 1, R, W), lambda k, r: (0, k, r, 0)),
            pl.BlockSpec((4, 1, R, W), lambda k, r: (0, k, r, 0)),
        ],
        out_shape=[
            jax.ShapeDtypeStruct((4, K, H, W), jnp.int32),
            jax.ShapeDtypeStruct((4, K, H, W), jnp.float32),
        ],
    )(off)


def _sc_gather_blend(table, idx, wgt):
    mesh = plsc.VectorSubcoreMesh(core_axis_name="c", subcore_axis_name="s")

    @functools.partial(
        pl.kernel,
        out_type=jax.ShapeDtypeStruct((N, C), jnp.float32),
        mesh=mesh,
        scratch_types=[
            pltpu.VMEM((4, CH), jnp.int32),
            pltpu.VMEM((4, CH), jnp.float32),
            pltpu.VMEM((4, CH, C), jnp.float32),
            pltpu.VMEM((CH, C), jnp.float32),
            pltpu.SemaphoreType.DMA,
        ],
        compiler_params=pltpu.CompilerParams(use_tc_tiling_on_sc=False),
    )
    def run(table_hbm, idx_hbm, wgt_hbm, o_hbm, idx_v, wgt_v, rows_v, out_v, sem):
        wid = lax.axis_index("s") * 2 + lax.axis_index("c")
        base = wid * NP

        def chunk_body(ci, carry):
            q0 = base + ci * CH
            pltpu.sync_copy(idx_hbm.at[:, pl.ds(q0, CH)], idx_v)
            pltpu.sync_copy(wgt_hbm.at[:, pl.ds(q0, CH)], wgt_v)
            cps = [pltpu.async_copy(table_hbm.at[idx_v.at[j]], rows_v.at[j], sem)
                   for j in range(4)]
            for cp in cps:
                cp.wait()

            def grp_body(g, rc):
                r0 = g * 16
                wv = [wgt_v[j, pl.ds(r0, 16)] for j in range(4)]
                for e in range(16):
                    r = r0 + e
                    for v in range(NV):
                        sl = pl.ds(v * 16, 16)
                        acc = rows_v[0, r, sl] * wv[0][e]
                        acc = acc + rows_v[1, r, sl] * wv[1][e]
                        acc = acc + rows_v[2, r, sl] * wv[2][e]
                        acc = acc + rows_v[3, r, sl] * wv[3][e]
                        out_v[r, sl] = acc
                return rc

            lax.fori_loop(0, CH // 16, grp_body, 0)
            pltpu.sync_copy(out_v, o_hbm.at[pl.ds(q0, CH)])
            return carry

        lax.fori_loop(0, NCHUNK, chunk_body, 0)

    return run(table, idx, wgt)


def _transpose_out(o1):
    PB = 512
    NPB = P // PB

    def body(x_ref, o_ref):
        o_ref[...] = x_ref[0].T

    return pl.pallas_call(
        body,
        grid=(K, NPB),
        in_specs=[pl.BlockSpec((1, PB, C), lambda k, j: (k, j, 0))],
        out_specs=pl.BlockSpec((C, PB), lambda k, j: (0, k * NPB + j)),
        out_shape=jax.ShapeDtypeStruct((C, N), jnp.float32),
    )(o1)


def kernel(input, offset):
    inp2d = input.reshape(C, P)
    off = offset.reshape(2 * K, H, W)
    table = _transpose_in(inp2d)
    idx, wgt = _prep(off)
    o1 = _sc_gather_blend(table, idx.reshape(4, N), wgt.reshape(4, N))
    o2 = _transpose_out(o1.reshape(K, P, C))
    return o2.reshape(1, C * K, P)
